# Initial kernel scaffold; baseline (speedup 1.0000x reference)
import jax
import jax.numpy as jnp
from jax import lax
from jax.experimental import pallas as pl
from jax.experimental.pallas import tpu as pltpu
from jax.experimental.pallas import tpu_sc as plsc

NN = 10000
EE = 160000
NC = 2
NS = 16
NW = NC * NS
RPT = NN // NS
CH = 200

_f32 = jnp.float32


def _deg_body(src_hbm, dst_hbm, w10_hbm, w01_hbm, zero2_hbm, out_hbm,
              idx_v, v10_v, v01_v, acc_sh):
    c = lax.axis_index("c")
    s = lax.axis_index("s")
    wid = c * NS + s
    pltpu.sync_copy(zero2_hbm.at[pl.ds(s * RPT, RPT), :],
                    acc_sh.at[pl.ds(s * RPT, RPT), :])
    pltpu.sync_copy(w10_hbm, v10_v)
    pltpu.sync_copy(w01_hbm, v01_v)
    plsc.subcore_barrier()
    epw = EE // NW

    def chunk(k, carry):
        base = wid * epw + k * CH
        pltpu.sync_copy(src_hbm.at[pl.ds(base, CH)], idx_v)
        pltpu.sync_copy(v10_v, acc_sh.at[idx_v], add=True)
        pltpu.sync_copy(dst_hbm.at[pl.ds(base, CH)], idx_v)
        pltpu.sync_copy(v01_v, acc_sh.at[idx_v], add=True)
        return carry

    lax.fori_loop(0, epw // CH, chunk, 0)
    plsc.subcore_barrier()
    pltpu.sync_copy(acc_sh.at[pl.ds(s * RPT, RPT), :],
                    out_hbm.at[c, pl.ds(s * RPT, RPT), :])


def _gcn_body(src_hbm, dst_hbm, w_hbm, hg_hbm, zero_hbm, out_hbm,
              idxs_v, idxd_v, w_v, rows_v, sem, acc_sh):
    c = lax.axis_index("c")
    s = lax.axis_index("s")
    wid = c * NS + s
    pltpu.sync_copy(zero_hbm.at[pl.ds(s * RPT, RPT), :],
                    acc_sh.at[pl.ds(s * RPT, RPT), :])
    plsc.subcore_barrier()
    epw = EE // NW

    def chunk(k, carry):
        base = wid * epw + k * CH
        pltpu.sync_copy(src_hbm.at[pl.ds(base, CH)], idxs_v)
        pltpu.sync_copy(dst_hbm.at[pl.ds(base, CH)], idxd_v)
        pltpu.sync_copy(w_hbm.at[pl.ds(base, CH)], w_v)
        pltpu.async_copy(hg_hbm.at[idxs_v], rows_v, sem).wait()

        def edge(i, c2):
            wv = plsc.load_gather(w_v, [jnp.full((16,), i, jnp.int32)])
            for j in range(12):
                rows_v[i, pl.ds(16 * j, 16)] = rows_v[i, pl.ds(16 * j, 16)] * wv
            return c2

        lax.fori_loop(0, CH, edge, 0)
        pltpu.sync_copy(rows_v, acc_sh.at[idxd_v], add=True)
        return carry

    lax.fori_loop(0, epw // CH, chunk, 0)
    plsc.subcore_barrier()
    pltpu.sync_copy(acc_sh.at[pl.ds(s * RPT, RPT), :],
                    out_hbm.at[c, pl.ds(s * RPT, RPT), :])


def _gat_body(src_hbm, dst_hbm, ew_hbm, h2k_hbm, h2v_hbm,
              esk_hbm, edk_hbm, esv_hbm, edv_hbm,
              zero64_hbm, zero1_hbm, outa_hbm, outd_hbm,
              idxs_v, idxd_v, es_v, ed_v, ew_v, coef_v, ex1_v, rows_v,
              tmp16_v, maxwt_v, sem, agg_sh, den_sh, maxw_sh):
    c = lax.axis_index("c")
    s = lax.axis_index("s")
    pltpu.sync_copy(zero64_hbm.at[pl.ds(s * RPT, RPT), :],
                    agg_sh.at[pl.ds(s * RPT, RPT), :])
    pltpu.sync_copy(zero1_hbm.at[pl.ds(s * RPT, RPT), :],
                    den_sh.at[pl.ds(s * RPT, RPT), :])
    plsc.subcore_barrier()
    ept = EE // NS
    nch = ept // CH
    iota = lax.iota(jnp.int32, 16)
    zer16 = jnp.zeros((16,), jnp.int32)

    def run(es_hbm, ed_hbm, h2_hbm):
        def mx_chunk(k, m):
            base = s * ept + k * CH
            pltpu.sync_copy(src_hbm.at[pl.ds(base, CH)], idxs_v)
            pltpu.sync_copy(dst_hbm.at[pl.ds(base, CH)], idxd_v)
            pltpu.async_copy(es_hbm.at[idxs_v], es_v, sem).wait()
            pltpu.async_copy(ed_hbm.at[idxd_v], ed_v, sem).wait()

            def inner(t, m2):
                sl = pl.ds(t * 16, 16)
                z = es_v[sl] + ed_v[sl]
                e = jnp.where(z > 0, z, 0.2 * z)
                return jnp.maximum(m2, e)

            return lax.fori_loop(0, CH // 16, inner, m)

        m = lax.fori_loop(0, nch, mx_chunk,
                          jnp.full((16,), -jnp.inf, _f32))
        tmp16_v[...] = m
        pltpu.sync_copy(tmp16_v, maxw_sh.at[s])
        plsc.subcore_barrier()
        pltpu.sync_copy(maxw_sh, maxwt_v)
        mall = maxwt_v[0, pl.ds(0, 16)]
        for r in range(1, NS):
            mall = jnp.maximum(mall, maxwt_v[r, pl.ds(0, 16)])
        mshift = jnp.max(mall)

        def chunk(k, carry):
            base = s * ept + k * CH
            pltpu.sync_copy(src_hbm.at[pl.ds(base, CH)], idxs_v)
            pltpu.sync_copy(dst_hbm.at[pl.ds(base, CH)], idxd_v)
            pltpu.sync_copy(ew_hbm.at[pl.ds(base, CH)], ew_v)
            pltpu.async_copy(es_hbm.at[idxs_v], es_v, sem).wait()
            pltpu.async_copy(ed_hbm.at[idxd_v], ed_v, sem).wait()
            pltpu.async_copy(h2_hbm.at[idxs_v], rows_v, sem).wait()

            def vec(t, c2):
                sl = pl.ds(t * 16, 16)
                z = es_v[sl] + ed_v[sl]
                e = jnp.where(z > 0, z, 0.2 * z)
                ex = jnp.exp(e - mshift)
                coef_v[sl] = ex * ew_v[sl]
                plsc.store_scatter(ex1_v, [iota + t * 16, zer16], ex)
                return c2

            lax.fori_loop(0, CH // 16, vec, 0)

            def edge(i, c2):
                wv = plsc.load_gather(coef_v, [jnp.full((16,), i, jnp.int32)])
                for j in range(4):
                    rows_v[i, pl.ds(16 * j, 16)] = (
                        rows_v[i, pl.ds(16 * j, 16)] * wv)
                return c2

            lax.fori_loop(0, CH, edge, 0)
            pltpu.sync_copy(rows_v, agg_sh.at[idxd_v], add=True)
            pltpu.sync_copy(ex1_v, den_sh.at[idxd_v], add=True)
            return carry

        lax.fori_loop(0, nch, chunk, 0)

    @pl.when(c == 0)
    def _():
        run(esk_hbm, edk_hbm, h2k_hbm)

    @pl.when(c == 1)
    def _():
        run(esv_hbm, edv_hbm, h2v_hbm)

    plsc.subcore_barrier()
    pltpu.sync_copy(agg_sh.at[pl.ds(s * RPT, RPT), :],
                    outa_hbm.at[c, pl.ds(s * RPT, RPT), :])
    pltpu.sync_copy(den_sh.at[pl.ds(s * RPT, RPT), :],
                    outd_hbm.at[c, pl.ds(s * RPT, RPT), :])


def _pre_body(q_ref, k_ref, v_ref, dg0_ref, dg1_ref,
              wq_ref, wk1_ref, wv1_ref, wk2_ref, wv2_ref,
              alk_ref, ark_ref, alv_ref, arv_ref,
              hg_ref, h2k_ref, h2v_ref, esk_ref, edk_ref, esv_ref, edv_ref):
    deg = dg0_ref[0] + dg1_ref[0]
    no = lax.rsqrt(jnp.clip(deg[:, 0:1], 1.0, None))
    hq = jnp.dot(q_ref[...], wq_ref[...], preferred_element_type=_f32)
    hk1 = jnp.dot(k_ref[...], wk1_ref[...], preferred_element_type=_f32)
    hv1 = jnp.dot(k_ref[...], wv1_ref[...], preferred_element_type=_f32)
    hg_ref[...] = jnp.concatenate([hq, hk1, hv1], axis=1) * no
    h2k = jnp.dot(v_ref[...], wk2_ref[...], preferred_element_type=_f32)
    h2v = jnp.dot(v_ref[...], wv2_ref[...], preferred_element_type=_f32)
    h2k_ref[...] = h2k
    h2v_ref[...] = h2v
    esk_ref[...] = jnp.dot(h2k, alk_ref[...], preferred_element_type=_f32)
    edk_ref[...] = jnp.dot(h2k, ark_ref[...], preferred_element_type=_f32)
    esv_ref[...] = jnp.dot(h2v, alv_ref[...], preferred_element_type=_f32)
    edv_ref[...] = jnp.dot(h2v, arv_ref[...], preferred_element_type=_f32)


def _post_body(a0_ref, a1_ref, ak2_ref, av2_ref, dk_ref, dv_ref,
               dg0_ref, dg1_ref, bq_ref, bk1_ref, bv1_ref, bk2_ref, bv2_ref,
               wh_ref, bh_ref, y_ref):
    agg = a0_ref[0] + a1_ref[0]
    deg = dg0_ref[0] + dg1_ref[0]
    ni = lax.rsqrt(jnp.clip(deg[:, 1:2], 1.0, None))
    q = agg[:, 0:64] * ni + bq_ref[...]
    k1 = agg[:, 64:128] * ni + bk1_ref[...]
    v1 = agg[:, 128:192] * ni + bv1_ref[...]
    denk = dk_ref[0]
    denv = dv_ref[0]
    k2 = ak2_ref[0] / jnp.where(denk > 0, denk, 1.0) + bk2_ref[...]
    v2 = av2_ref[0] / jnp.where(denv > 0, denv, 1.0) + bv2_ref[...]
    kv1 = jnp.sum(q * k1, axis=1, keepdims=True)
    kv2 = jnp.sum(q * k2, axis=1, keepdims=True)
    mx = jnp.maximum(kv1, kv2)
    e1 = jnp.exp(kv1 - mx)
    e2 = jnp.exp(kv2 - mx)
    ssum = e1 + e2
    out = (e1 / ssum) * v1 + (e2 / ssum) * v2
    wh = wh_ref[...]
    whf = wh[0:64] + wh[64:128] + wh[128:192] + wh[192:256]
    y_ref[...] = jnp.dot(out, whf, preferred_element_type=_f32) + bh_ref[...]


def _sc_mesh():
    return plsc.VectorSubcoreMesh(core_axis_name="c", subcore_axis_name="s",
                                  num_cores=NC, num_subcores=NS)


def kernel(Q, K, V, sg_edge_index, edfg_edge_index, sgFeat, edfgFeat,
           Wq, bq, Wk1, bk1, Wv1, bv1,
           Wk2, alk2, ark2, bk2, Wv2, alv2, arv2, bv2,
           Wh, bh):
    src_s = sg_edge_index[0]
    dst_s = sg_edge_index[1]
    src_e = edfg_edge_index[0]
    dst_e = edfg_edge_index[1]

    w10 = jnp.tile(jnp.array([[1.0, 0.0]], _f32), (CH, 1))
    w01 = jnp.tile(jnp.array([[0.0, 1.0]], _f32), (CH, 1))
    zeros2 = jnp.zeros((NN, 2), _f32)
    zeros192 = jnp.zeros((NN, 192), _f32)
    zeros64 = jnp.zeros((NN, 64), _f32)
    zeros1 = jnp.zeros((NN, 1), _f32)

    deg_call = pl.kernel(
        _deg_body,
        out_type=jax.ShapeDtypeStruct((NC, NN, 2), _f32),
        mesh=_sc_mesh(),
        scratch_types=[
            pltpu.VMEM((CH,), jnp.int32),
            pltpu.VMEM((CH, 2), _f32),
            pltpu.VMEM((CH, 2), _f32),
            pltpu.VMEM_SHARED((NN, 2), _f32),
        ],
    )
    degp = deg_call(src_s, dst_s, w10, w01, zeros2)

    BN = 1000
    grid = (NN // BN,)

    pre_call = pl.pallas_call(
        _pre_body,
        grid=grid,
        in_specs=[
            pl.BlockSpec((BN, 256), lambda i: (i, 0)),
            pl.BlockSpec((BN, 256), lambda i: (i, 0)),
            pl.BlockSpec((BN, 256), lambda i: (i, 0)),
            pl.BlockSpec((1, BN, 2), lambda i: (0, i, 0)),
            pl.BlockSpec((1, BN, 2), lambda i: (1, i, 0)),
            pl.BlockSpec((256, 64), lambda i: (0, 0)),
            pl.BlockSpec((256, 64), lambda i: (0, 0)),
            pl.BlockSpec((256, 64), lambda i: (0, 0)),
            pl.BlockSpec((256, 64), lambda i: (0, 0)),
            pl.BlockSpec((256, 64), lambda i: (0, 0)),
            pl.BlockSpec((64, 1), lambda i: (0, 0)),
            pl.BlockSpec((64, 1), lambda i: (0, 0)),
            pl.BlockSpec((64, 1), lambda i: (0, 0)),
            pl.BlockSpec((64, 1), lambda i: (0, 0)),
        ],
        out_specs=[
            pl.BlockSpec((BN, 192), lambda i: (i, 0)),
            pl.BlockSpec((BN, 64), lambda i: (i, 0)),
            pl.BlockSpec((BN, 64), lambda i: (i, 0)),
            pl.BlockSpec((BN, 1), lambda i: (i, 0)),
            pl.BlockSpec((BN, 1), lambda i: (i, 0)),
            pl.BlockSpec((BN, 1), lambda i: (i, 0)),
            pl.BlockSpec((BN, 1), lambda i: (i, 0)),
        ],
        out_shape=[
            jax.ShapeDtypeStruct((NN, 192), _f32),
            jax.ShapeDtypeStruct((NN, 64), _f32),
            jax.ShapeDtypeStruct((NN, 64), _f32),
            jax.ShapeDtypeStruct((NN, 1), _f32),
            jax.ShapeDtypeStruct((NN, 1), _f32),
            jax.ShapeDtypeStruct((NN, 1), _f32),
            jax.ShapeDtypeStruct((NN, 1), _f32),
        ],
    )
    hg, h2k, h2v, esk, edk, esv, edv = pre_call(
        Q, K, V, degp, degp, Wq, Wk1, Wv1, Wk2, Wv2,
        alk2.reshape(64, 1), ark2.reshape(64, 1),
        alv2.reshape(64, 1), arv2.reshape(64, 1))

    gcn_call = pl.kernel(
        _gcn_body,
        out_type=jax.ShapeDtypeStruct((NC, NN, 192), _f32),
        mesh=_sc_mesh(),
        scratch_types=[
            pltpu.VMEM((CH,), jnp.int32),
            pltpu.VMEM((CH,), jnp.int32),
            pltpu.VMEM((CH,), _f32),
            pltpu.VMEM((CH, 192), _f32),
            pltpu.SemaphoreType.DMA,
            pltpu.VMEM_SHARED((NN, 192), _f32),
        ],
    )
    aggp = gcn_call(src_s, dst_s, sgFeat, hg, zeros192)

    gat_call = pl.kernel(
        _gat_body,
        out_type=(
            jax.ShapeDtypeStruct((NC, NN, 64), _f32),
            jax.ShapeDtypeStruct((NC, NN, 1), _f32),
        ),
        mesh=_sc_mesh(),
        scratch_types=[
            pltpu.VMEM((CH,), jnp.int32),
            pltpu.VMEM((CH,), jnp.int32),
            pltpu.VMEM((CH,), _f32),
            pltpu.VMEM((CH,), _f32),
            pltpu.VMEM((CH,), _f32),
            pltpu.VMEM((CH,), _f32),
            pltpu.VMEM((CH, 1), _f32),
            pltpu.VMEM((CH, 64), _f32),
            pltpu.VMEM((16,), _f32),
            pltpu.VMEM((NS, 16), _f32),
            pltpu.SemaphoreType.DMA,
            pltpu.VMEM_SHARED((NN, 64), _f32),
            pltpu.VMEM_SHARED((NN, 1), _f32),
            pltpu.VMEM_SHARED((NS, 16), _f32),
        ],
    )
    agg2, den = gat_call(src_e, dst_e, edfgFeat, h2k, h2v,
                         esk.reshape(NN), edk.reshape(NN),
                         esv.reshape(NN), edv.reshape(NN),
                         zeros64, zeros1)

    post_call = pl.pallas_call(
        _post_body,
        grid=grid,
        in_specs=[
            pl.BlockSpec((1, BN, 192), lambda i: (0, i, 0)),
            pl.BlockSpec((1, BN, 192), lambda i: (1, i, 0)),
            pl.BlockSpec((1, BN, 64), lambda i: (0, i, 0)),
            pl.BlockSpec((1, BN, 64), lambda i: (1, i, 0)),
            pl.BlockSpec((1, BN, 1), lambda i: (0, i, 0)),
            pl.BlockSpec((1, BN, 1), lambda i: (1, i, 0)),
            pl.BlockSpec((1, BN, 2), lambda i: (0, i, 0)),
            pl.BlockSpec((1, BN, 2), lambda i: (1, i, 0)),
            pl.BlockSpec((1, 64), lambda i: (0, 0)),
            pl.BlockSpec((1, 64), lambda i: (0, 0)),
            pl.BlockSpec((1, 64), lambda i: (0, 0)),
            pl.BlockSpec((1, 64), lambda i: (0, 0)),
            pl.BlockSpec((1, 64), lambda i: (0, 0)),
            pl.BlockSpec((256, 256), lambda i: (0, 0)),
            pl.BlockSpec((1, 256), lambda i: (0, 0)),
        ],
        out_specs=pl.BlockSpec((BN, 256), lambda i: (i, 0)),
        out_shape=jax.ShapeDtypeStruct((NN, 256), _f32),
    )
    y = post_call(aggp, aggp, agg2, agg2, den, den, degp, degp,
                  bq.reshape(1, 64), bk1.reshape(1, 64), bv1.reshape(1, 64),
                  bk2.reshape(1, 64), bv2.reshape(1, 64),
                  Wh, bh.reshape(1, 256))
    return y


# speedup vs baseline: 8.4074x; 8.4074x over previous
import jax
import jax.numpy as jnp
from jax import lax
from jax.experimental import pallas as pl
from jax.experimental.pallas import tpu as pltpu
from jax.experimental.pallas import tpu_sc as plsc

NN = 10000
NP = 10240
EE = 160000
EP = 163840
NC = 2
NS = 16
NW = NC * NS
RPT = NP // NS
CH = 128

_f32 = jnp.float32

_GDN = lax.GatherDimensionNumbers(
    offset_dims=(), collapsed_slice_dims=(0,), start_index_map=(0,))


def _bcast(vec, lane):
    idx = jnp.full((16, 1), lane, jnp.int32)
    return lax.gather(vec, idx, _GDN, (1,),
                      mode=lax.GatherScatterMode.PROMISE_IN_BOUNDS)


def _deg_body(src_hbm, dst_hbm, w10_hbm, w01_hbm, zero2_hbm, out_hbm,
              idx_v, v10_v, v01_v, acc_sh):
    c = lax.axis_index("c")
    s = lax.axis_index("s")
    wid = c * NS + s
    pltpu.sync_copy(zero2_hbm.at[pl.ds(s * RPT, RPT), :],
                    acc_sh.at[pl.ds(s * RPT, RPT), :])
    pltpu.sync_copy(w10_hbm, v10_v)
    pltpu.sync_copy(w01_hbm, v01_v)
    plsc.subcore_barrier()
    epw = EP // NW

    def chunk(k, carry):
        base = wid * epw + k * CH
        pltpu.sync_copy(src_hbm.at[pl.ds(base, CH)], idx_v)
        pltpu.sync_copy(v10_v, acc_sh.at[idx_v], add=True)
        pltpu.sync_copy(dst_hbm.at[pl.ds(base, CH)], idx_v)
        pltpu.sync_copy(v01_v, acc_sh.at[idx_v], add=True)
        return carry

    lax.fori_loop(0, epw // CH, chunk, 0)
    plsc.subcore_barrier()
    pltpu.sync_copy(acc_sh.at[pl.ds(s * RPT, RPT), :],
                    out_hbm.at[c, pl.ds(s * RPT, RPT), :])


def _gcn_body(src_hbm, dst_hbm, w_hbm, hga_hbm, hgb_hbm, zero_hbm, out_hbm,
              idxs_v, idxd_v, w_v, rows_v, sem, acc_sh):
    c = lax.axis_index("c")
    s = lax.axis_index("s")
    pltpu.sync_copy(zero_hbm.at[pl.ds(s * RPT, RPT), :],
                    acc_sh.at[pl.ds(s * RPT, RPT), :])
    plsc.subcore_barrier()
    ept = EP // NS

    def run(hg_hbm):
        def chunk(k, carry):
            base = s * ept + k * CH
            pltpu.sync_copy(src_hbm.at[pl.ds(base, CH)], idxs_v)
            pltpu.sync_copy(dst_hbm.at[pl.ds(base, CH)], idxd_v)
            pltpu.sync_copy(w_hbm.at[pl.ds(base, CH)], w_v)
            pltpu.async_copy(hg_hbm.at[idxs_v], rows_v, sem).wait()

            def group(t, c2):
                gw = w_v[pl.ds(t * 16, 16)]
                for l in range(16):
                    wl = _bcast(gw, l)
                    i = t * 16 + l
                    for j in range(6):
                        rows_v[i, pl.ds(16 * j, 16)] = (
                            rows_v[i, pl.ds(16 * j, 16)] * wl)
                return c2

            lax.fori_loop(0, CH // 16, group, 0)
            pltpu.sync_copy(rows_v, acc_sh.at[idxd_v], add=True)
            return carry

        lax.fori_loop(0, ept // CH, chunk, 0)

    @pl.when(c == 0)
    def _():
        run(hga_hbm)

    @pl.when(c == 1)
    def _():
        run(hgb_hbm)

    plsc.subcore_barrier()
    pltpu.sync_copy(acc_sh.at[pl.ds(s * RPT, RPT), :],
                    out_hbm.at[c, pl.ds(s * RPT, RPT), :])


def _gat_body(src_hbm, dst_hbm, ew_hbm, h2k_hbm, h2v_hbm,
              esk_hbm, edk_hbm, esv_hbm, edv_hbm,
              zero64_hbm, zero1_hbm, outa_hbm, outd_hbm,
              idxs_v, idxd_v, es_v, ed_v, ew_v, coef_v, ex1_v, rows_v,
              tmp16_v, maxwt_v, sem, agg_sh, den_sh, maxw_sh):
    c = lax.axis_index("c")
    s = lax.axis_index("s")
    pltpu.sync_copy(zero64_hbm.at[pl.ds(s * RPT, RPT), :],
                    agg_sh.at[pl.ds(s * RPT, RPT), :])
    pltpu.sync_copy(zero1_hbm.at[pl.ds(s * RPT, RPT)],
                    den_sh.at[pl.ds(s * RPT, RPT)])
    plsc.subcore_barrier()
    ept = EP // NS
    nch = ept // CH
    iota = lax.iota(jnp.int32, 16)

    def run(es_hbm, ed_hbm, h2_hbm):
        def mx_chunk(k, m):
            base = s * ept + k * CH
            pltpu.sync_copy(src_hbm.at[pl.ds(base, CH)], idxs_v)
            pltpu.sync_copy(dst_hbm.at[pl.ds(base, CH)], idxd_v)
            pltpu.async_copy(es_hbm.at[idxs_v], es_v, sem).wait()
            pltpu.async_copy(ed_hbm.at[idxd_v], ed_v, sem).wait()

            def inner(t, m2):
                sl = pl.ds(t * 16, 16)
                z = es_v[sl] + ed_v[sl]
                e = jnp.where(z > 0, z, 0.2 * z)
                return jnp.maximum(m2, e)

            return lax.fori_loop(0, CH // 16, inner, m)

        m = lax.fori_loop(0, nch, mx_chunk,
                          jnp.full((16,), -jnp.inf, _f32))
        tmp16_v[...] = m
        pltpu.sync_copy(tmp16_v, maxw_sh.at[s * 8])
        plsc.subcore_barrier()
        pltpu.sync_copy(maxw_sh, maxwt_v)
        mall = maxwt_v[0, pl.ds(0, 16)]
        for r in range(1, NS):
            mall = jnp.maximum(mall, maxwt_v[r * 8, pl.ds(0, 16)])
        for sh in (1, 2, 4, 8):
            rolled = lax.gather(
                mall, ((iota + sh) & 15).reshape(16, 1), _GDN,
                slice_sizes=(1,),
                mode=lax.GatherScatterMode.PROMISE_IN_BOUNDS)
            mall = jnp.maximum(mall, rolled)
        mshift = mall

        def chunk(k, carry):
            base = s * ept + k * CH
            pltpu.sync_copy(src_hbm.at[pl.ds(base, CH)], idxs_v)
            pltpu.sync_copy(dst_hbm.at[pl.ds(base, CH)], idxd_v)
            pltpu.sync_copy(ew_hbm.at[pl.ds(base, CH)], ew_v)
            pltpu.async_copy(es_hbm.at[idxs_v], es_v, sem).wait()
            pltpu.async_copy(ed_hbm.at[idxd_v], ed_v, sem).wait()
            pltpu.async_copy(h2_hbm.at[idxs_v], rows_v, sem).wait()

            def vec(t, c2):
                sl = pl.ds(t * 16, 16)
                z = es_v[sl] + ed_v[sl]
                e = jnp.where(z > 0, z, 0.2 * z)
                ex = jnp.exp(e - mshift)
                coef_v[sl] = ex * ew_v[sl]
                ex1_v[sl] = ex
                return c2

            lax.fori_loop(0, CH // 16, vec, 0)

            def group(t, c2):
                gw = coef_v[pl.ds(t * 16, 16)]
                for l in range(16):
                    wl = _bcast(gw, l)
                    i = t * 16 + l
                    for j in range(4):
                        rows_v[i, pl.ds(16 * j, 16)] = (
                            rows_v[i, pl.ds(16 * j, 16)] * wl)
                return c2

            lax.fori_loop(0, CH // 16, group, 0)
            pltpu.sync_copy(rows_v, agg_sh.at[idxd_v], add=True)
            pltpu.sync_copy(ex1_v, den_sh.at[idxd_v], add=True)
            return carry

        lax.fori_loop(0, nch, chunk, 0)

    @pl.when(c == 0)
    def _():
        run(esk_hbm, edk_hbm, h2k_hbm)

    @pl.when(c == 1)
    def _():
        run(esv_hbm, edv_hbm, h2v_hbm)

    plsc.subcore_barrier()
    pltpu.sync_copy(agg_sh.at[pl.ds(s * RPT, RPT), :],
                    outa_hbm.at[c, pl.ds(s * RPT, RPT), :])
    pltpu.sync_copy(den_sh.at[pl.ds(s * RPT, RPT)],
                    outd_hbm.at[pl.ds(c * NP + s * RPT, RPT)])


def _pre_body(q_ref, k_ref, v_ref, dg0_ref, dg1_ref,
              wq_ref, wk1_ref, wv1_ref, wk2_ref, wv2_ref,
              alk_ref, ark_ref, alv_ref, arv_ref,
              hga_ref, hgb_ref, h2k_ref, h2v_ref,
              esk_ref, edk_ref, esv_ref, edv_ref):
    deg = dg0_ref[0] + dg1_ref[0]
    no = lax.rsqrt(jnp.clip(deg[:, 0:1], 1.0, None))
    hq = jnp.dot(q_ref[...], wq_ref[...], preferred_element_type=_f32)
    hk1 = jnp.dot(k_ref[...], wk1_ref[...], preferred_element_type=_f32)
    hv1 = jnp.dot(k_ref[...], wv1_ref[...], preferred_element_type=_f32)
    hga_ref[...] = jnp.concatenate([hq, hk1[:, 0:32]], axis=1) * no
    hgb_ref[...] = jnp.concatenate([hk1[:, 32:64], hv1], axis=1) * no
    h2k = jnp.dot(v_ref[...], wk2_ref[...], preferred_element_type=_f32)
    h2v = jnp.dot(v_ref[...], wv2_ref[...], preferred_element_type=_f32)
    h2k_ref[...] = h2k
    h2v_ref[...] = h2v
    esk_ref[...] = jnp.dot(h2k, alk_ref[...], preferred_element_type=_f32)
    edk_ref[...] = jnp.dot(h2k, ark_ref[...], preferred_element_type=_f32)
    esv_ref[...] = jnp.dot(h2v, alv_ref[...], preferred_element_type=_f32)
    edv_ref[...] = jnp.dot(h2v, arv_ref[...], preferred_element_type=_f32)


def _post_body(a0_ref, a1_ref, ak2_ref, av2_ref, dk_ref, dv_ref,
               dg0_ref, dg1_ref, bq_ref, bk1_ref, bv1_ref, bk2_ref, bv2_ref,
               wh_ref, bh_ref, y_ref):
    alo = a0_ref[0]
    ahi = a1_ref[0]
    deg = dg0_ref[0] + dg1_ref[0]
    ni = lax.rsqrt(jnp.clip(deg[:, 1:2], 1.0, None))
    q = alo[:, 0:64] * ni + bq_ref[...]
    k1 = (jnp.concatenate([alo[:, 64:96], ahi[:, 0:32]], axis=1) * ni
          + bk1_ref[...])
    v1 = ahi[:, 32:96] * ni + bv1_ref[...]
    denk = dk_ref[0]
    denv = dv_ref[0]
    k2 = ak2_ref[0] / jnp.where(denk > 0, denk, 1.0) + bk2_ref[...]
    v2 = av2_ref[0] / jnp.where(denv > 0, denv, 1.0) + bv2_ref[...]
    kv1 = jnp.sum(q * k1, axis=1, keepdims=True)
    kv2 = jnp.sum(q * k2, axis=1, keepdims=True)
    mx = jnp.maximum(kv1, kv2)
    e1 = jnp.exp(kv1 - mx)
    e2 = jnp.exp(kv2 - mx)
    ssum = e1 + e2
    out = (e1 / ssum) * v1 + (e2 / ssum) * v2
    wh = wh_ref[...]
    whf = wh[0:64] + wh[64:128] + wh[128:192] + wh[192:256]
    y_ref[...] = jnp.dot(out, whf, preferred_element_type=_f32) + bh_ref[...]


def _sc_mesh():
    return plsc.VectorSubcoreMesh(core_axis_name="c", subcore_axis_name="s",
                                  num_cores=NC, num_subcores=NS)


def kernel(Q, K, V, sg_edge_index, edfg_edge_index, sgFeat, edfgFeat,
           Wq, bq, Wk1, bk1, Wv1, bv1,
           Wk2, alk2, ark2, bk2, Wv2, alv2, arv2, bv2,
           Wh, bh):
    npad = EP - EE
    pad0 = jnp.zeros((npad,), jnp.int32)
    padn = jnp.full((npad,), NN, jnp.int32)
    padw = jnp.zeros((npad,), _f32)
    src_s = jnp.concatenate([sg_edge_index[0], pad0])
    srcdeg_s = jnp.concatenate([sg_edge_index[0], padn])
    dst_s = jnp.concatenate([sg_edge_index[1], padn])
    sgw = jnp.concatenate([sgFeat, padw])
    src_e = jnp.concatenate([edfg_edge_index[0], pad0])
    dst_e = jnp.concatenate([edfg_edge_index[1], padn])
    edw = jnp.concatenate([edfgFeat, padw])

    w10 = jnp.tile(jnp.array([[1.0] + [0.0] * 15], _f32), (CH, 1))
    w01 = jnp.tile(jnp.array([[0.0, 1.0] + [0.0] * 14], _f32), (CH, 1))
    zeros2 = jnp.zeros((NP, 16), _f32)
    zeros96 = jnp.zeros((NP, 96), _f32)
    zeros64 = jnp.zeros((NP, 64), _f32)
    zeros1 = jnp.zeros((NP,), _f32)

    deg_call = pl.kernel(
        _deg_body,
        out_type=jax.ShapeDtypeStruct((NC, NP, 16), _f32),
        mesh=_sc_mesh(),
        compiler_params=pltpu.CompilerParams(use_tc_tiling_on_sc=False),
        scratch_types=[
            pltpu.VMEM((CH,), jnp.int32),
            pltpu.VMEM((CH, 16), _f32),
            pltpu.VMEM((CH, 16), _f32),
            pltpu.VMEM_SHARED((NP, 16), _f32),
        ],
    )
    degp = deg_call(srcdeg_s, dst_s, w10, w01, zeros2)

    BN = 1000
    grid = (NN // BN,)

    pre_call = pl.pallas_call(
        _pre_body,
        grid=grid,
        in_specs=[
            pl.BlockSpec((BN, 256), lambda i: (i, 0)),
            pl.BlockSpec((BN, 256), lambda i: (i, 0)),
            pl.BlockSpec((BN, 256), lambda i: (i, 0)),
            pl.BlockSpec((1, BN, 16), lambda i: (0, i, 0)),
            pl.BlockSpec((1, BN, 16), lambda i: (1, i, 0)),
            pl.BlockSpec((256, 64), lambda i: (0, 0)),
            pl.BlockSpec((256, 64), lambda i: (0, 0)),
            pl.BlockSpec((256, 64), lambda i: (0, 0)),
            pl.BlockSpec((256, 64), lambda i: (0, 0)),
            pl.BlockSpec((256, 64), lambda i: (0, 0)),
            pl.BlockSpec((64, 1), lambda i: (0, 0)),
            pl.BlockSpec((64, 1), lambda i: (0, 0)),
            pl.BlockSpec((64, 1), lambda i: (0, 0)),
            pl.BlockSpec((64, 1), lambda i: (0, 0)),
        ],
        out_specs=[
            pl.BlockSpec((BN, 96), lambda i: (i, 0)),
            pl.BlockSpec((BN, 96), lambda i: (i, 0)),
            pl.BlockSpec((BN, 64), lambda i: (i, 0)),
            pl.BlockSpec((BN, 64), lambda i: (i, 0)),
            pl.BlockSpec((BN, 1), lambda i: (i, 0)),
            pl.BlockSpec((BN, 1), lambda i: (i, 0)),
            pl.BlockSpec((BN, 1), lambda i: (i, 0)),
            pl.BlockSpec((BN, 1), lambda i: (i, 0)),
        ],
        out_shape=[
            jax.ShapeDtypeStruct((NN, 96), _f32),
            jax.ShapeDtypeStruct((NN, 96), _f32),
            jax.ShapeDtypeStruct((NN, 64), _f32),
            jax.ShapeDtypeStruct((NN, 64), _f32),
            jax.ShapeDtypeStruct((NN, 1), _f32),
            jax.ShapeDtypeStruct((NN, 1), _f32),
            jax.ShapeDtypeStruct((NN, 1), _f32),
            jax.ShapeDtypeStruct((NN, 1), _f32),
        ],
    )
    hga, hgb, h2k, h2v, esk, edk, esv, edv = pre_call(
        Q, K, V, degp, degp, Wq, Wk1, Wv1, Wk2, Wv2,
        alk2.reshape(64, 1), ark2.reshape(64, 1),
        alv2.reshape(64, 1), arv2.reshape(64, 1))

    gcn_call = pl.kernel(
        _gcn_body,
        out_type=jax.ShapeDtypeStruct((NC, NP, 96), _f32),
        mesh=_sc_mesh(),
        compiler_params=pltpu.CompilerParams(use_tc_tiling_on_sc=False),
        scratch_types=[
            pltpu.VMEM((CH,), jnp.int32),
            pltpu.VMEM((CH,), jnp.int32),
            pltpu.VMEM((CH,), _f32),
            pltpu.VMEM((CH, 96), _f32),
            pltpu.SemaphoreType.DMA,
            pltpu.VMEM_SHARED((NP, 96), _f32),
        ],
    )
    aggp = gcn_call(src_s, dst_s, sgw, hga, hgb, zeros96)

    gat_call = pl.kernel(
        _gat_body,
        out_type=(
            jax.ShapeDtypeStruct((NC, NP, 64), _f32),
            jax.ShapeDtypeStruct((NC * NP,), _f32),
        ),
        mesh=_sc_mesh(),
        compiler_params=pltpu.CompilerParams(use_tc_tiling_on_sc=False),
        scratch_types=[
            pltpu.VMEM((CH,), jnp.int32),
            pltpu.VMEM((CH,), jnp.int32),
            pltpu.VMEM((CH,), _f32),
            pltpu.VMEM((CH,), _f32),
            pltpu.VMEM((CH,), _f32),
            pltpu.VMEM((CH,), _f32),
            pltpu.VMEM((CH,), _f32),
            pltpu.VMEM((CH, 64), _f32),
            pltpu.VMEM((16,), _f32),
            pltpu.VMEM((NS * 8, 16), _f32),
            pltpu.SemaphoreType.DMA,
            pltpu.VMEM_SHARED((NP, 64), _f32),
            pltpu.VMEM_SHARED((NP,), _f32),
            pltpu.VMEM_SHARED((NS * 8, 16), _f32),
        ],
    )
    padt = jnp.zeros((NP - NN,), _f32)
    edk_p = jnp.concatenate([edk.reshape(NN), padt])
    edv_p = jnp.concatenate([edv.reshape(NN), padt])
    agg2, den_flat = gat_call(src_e, dst_e, edw, h2k, h2v,
                              esk.reshape(NN), edk_p,
                              esv.reshape(NN), edv_p,
                              zeros64, zeros1)
    den = den_flat.reshape(NC, NP, 1)

    post_call = pl.pallas_call(
        _post_body,
        grid=grid,
        in_specs=[
            pl.BlockSpec((1, BN, 96), lambda i: (0, i, 0)),
            pl.BlockSpec((1, BN, 96), lambda i: (1, i, 0)),
            pl.BlockSpec((1, BN, 64), lambda i: (0, i, 0)),
            pl.BlockSpec((1, BN, 64), lambda i: (1, i, 0)),
            pl.BlockSpec((1, BN, 1), lambda i: (0, i, 0)),
            pl.BlockSpec((1, BN, 1), lambda i: (1, i, 0)),
            pl.BlockSpec((1, BN, 16), lambda i: (0, i, 0)),
            pl.BlockSpec((1, BN, 16), lambda i: (1, i, 0)),
            pl.BlockSpec((1, 64), lambda i: (0, 0)),
            pl.BlockSpec((1, 64), lambda i: (0, 0)),
            pl.BlockSpec((1, 64), lambda i: (0, 0)),
            pl.BlockSpec((1, 64), lambda i: (0, 0)),
            pl.BlockSpec((1, 64), lambda i: (0, 0)),
            pl.BlockSpec((256, 256), lambda i: (0, 0)),
            pl.BlockSpec((1, 256), lambda i: (0, 0)),
        ],
        out_specs=pl.BlockSpec((BN, 256), lambda i: (i, 0)),
        out_shape=jax.ShapeDtypeStruct((NN, 256), _f32),
    )
    y = post_call(aggp, aggp, agg2, agg2, den, den, degp, degp,
                  bq.reshape(1, 64), bk1.reshape(1, 64), bv1.reshape(1, 64),
                  bk2.reshape(1, 64), bv2.reshape(1, 64),
                  Wh, bh.reshape(1, 256))
    return y


# speedup vs baseline: 18.6107x; 2.2136x over previous
import jax
import jax.numpy as jnp
from jax import lax
from jax.experimental import pallas as pl
from jax.experimental.pallas import tpu as pltpu
from jax.experimental.pallas import tpu_sc as plsc

NN = 10000
NP = 10240
EE = 160000
EP = 163840
NC = 2
NS = 16
NW = NC * NS
RPT = NP // NS
CH = 128
NCHD = (EP // NW) // CH
NCHT = (EP // NS) // CH

_f32 = jnp.float32

_GDN = lax.GatherDimensionNumbers(
    offset_dims=(), collapsed_slice_dims=(0,), start_index_map=(0,))


def _bcast(vec, lane):
    idx = jnp.full((16, 1), lane, jnp.int32)
    return lax.gather(vec, idx, _GDN, (1,),
                      mode=lax.GatherScatterMode.PROMISE_IN_BOUNDS)


def _deg_body(srcr, dstr, w10_hbm, w01_hbm, zero_hbm, out_hbm,
              idxs2_v, idxd2_v, v10_v, v01_v, acc_sh):
    c = lax.axis_index("c")
    s = lax.axis_index("s")
    wid = c * NS + s
    pltpu.sync_copy(zero_hbm.at[pl.ds(s * RPT, RPT), :],
                    acc_sh.at[pl.ds(s * RPT, RPT), :])
    pltpu.sync_copy(w10_hbm, v10_v)
    pltpu.sync_copy(w01_hbm, v01_v)
    pltpu.sync_copy(srcr.at[pl.ds(wid * NCHD, NCHD), :], idxs2_v)
    pltpu.sync_copy(dstr.at[pl.ds(wid * NCHD, NCHD), :], idxd2_v)
    plsc.subcore_barrier()

    def chunk(k, carry):
        pltpu.sync_copy(v10_v, acc_sh.at[idxs2_v.at[k]], add=True)
        pltpu.sync_copy(v01_v, acc_sh.at[idxd2_v.at[k]], add=True)
        return carry

    lax.fori_loop(0, NCHD, chunk, 0)
    plsc.subcore_barrier()
    pltpu.sync_copy(acc_sh.at[pl.ds(s * RPT, RPT), :],
                    out_hbm.at[c, pl.ds(s * RPT, RPT), :])


def _gcn_body(srcr, dstr, wr, hga_hbm, hgb_hbm, zero_hbm, out_hbm,
              idxs2_v, idxd2_v, w2_v, rows0_v, rows1_v, sem0, sem1, acc_sh):
    c = lax.axis_index("c")
    s = lax.axis_index("s")
    pltpu.sync_copy(zero_hbm.at[pl.ds(s * RPT, RPT), :],
                    acc_sh.at[pl.ds(s * RPT, RPT), :])
    pltpu.sync_copy(srcr.at[pl.ds(s * NCHT, NCHT), :], idxs2_v)
    pltpu.sync_copy(dstr.at[pl.ds(s * NCHT, NCHT), :], idxd2_v)
    pltpu.sync_copy(wr.at[pl.ds(s * NCHT, NCHT), :], w2_v)
    plsc.subcore_barrier()

    def run(hg_hbm):
        pltpu.async_copy(hg_hbm.at[idxs2_v.at[0]], rows0_v, sem0)
        pltpu.async_copy(hg_hbm.at[idxs2_v.at[1]], rows1_v, sem1)

        def process(j, rows_v, sem):
            pltpu.make_async_copy(
                hg_hbm.at[idxs2_v.at[j]], rows_v, sem).wait()

            def group(t, c2):
                gw = w2_v[j, pl.ds(t * 16, 16)]
                for l in range(16):
                    wl = _bcast(gw, l)
                    i = t * 16 + l
                    for jj in range(6):
                        rows_v[i, pl.ds(16 * jj, 16)] = (
                            rows_v[i, pl.ds(16 * jj, 16)] * wl)
                return c2

            lax.fori_loop(0, CH // 16, group, 0)
            pltpu.sync_copy(rows_v, acc_sh.at[idxd2_v.at[j]], add=True)
            nj = j + 2

            @pl.when(nj < NCHT)
            def _():
                pltpu.async_copy(hg_hbm.at[idxs2_v.at[nj]], rows_v, sem)

        def body(m, carry):
            process(2 * m, rows0_v, sem0)
            process(2 * m + 1, rows1_v, sem1)
            return carry

        lax.fori_loop(0, NCHT // 2, body, 0)

    @pl.when(c == 0)
    def _():
        run(hga_hbm)

    @pl.when(c == 1)
    def _():
        run(hgb_hbm)

    plsc.subcore_barrier()
    pltpu.sync_copy(acc_sh.at[pl.ds(s * RPT, RPT), :],
                    out_hbm.at[c, pl.ds(s * RPT, RPT), :])


def _gat_body(srcr, dstr, ewr, h2k_hbm, h2v_hbm,
              esk_hbm, edk_hbm, esv_hbm, edv_hbm, mx_hbm,
              zero64_hbm, zero1_hbm, outa_hbm, outd_hbm,
              idxs2_v, idxd2_v, ew2_v, es0_v, es1_v, ed0_v, ed1_v,
              rows0_v, rows1_v, coef_v, ex1_v, mx_v, sem0, sem1,
              agg_sh, den_sh):
    c = lax.axis_index("c")
    s = lax.axis_index("s")
    pltpu.sync_copy(zero64_hbm.at[pl.ds(s * RPT, RPT), :],
                    agg_sh.at[pl.ds(s * RPT, RPT), :])
    pltpu.sync_copy(zero1_hbm.at[pl.ds(s * RPT, RPT)],
                    den_sh.at[pl.ds(s * RPT, RPT)])
    pltpu.sync_copy(srcr.at[pl.ds(s * NCHT, NCHT), :], idxs2_v)
    pltpu.sync_copy(dstr.at[pl.ds(s * NCHT, NCHT), :], idxd2_v)
    pltpu.sync_copy(ewr.at[pl.ds(s * NCHT, NCHT), :], ew2_v)
    pltpu.sync_copy(mx_hbm, mx_v)
    plsc.subcore_barrier()

    def run(es_hbm, ed_hbm, h2_hbm, r0, r1):
        zz = mx_v[r0, pl.ds(0, 16)] + mx_v[r1, pl.ds(0, 16)]
        shift = jnp.where(zz > 0, zz, 0.2 * zz)

        def fire(j, es_v, ed_v, rows_v, sem):
            pltpu.async_copy(es_hbm.at[idxs2_v.at[j]], es_v, sem)
            pltpu.async_copy(ed_hbm.at[idxd2_v.at[j]], ed_v, sem)
            pltpu.async_copy(h2_hbm.at[idxs2_v.at[j]], rows_v, sem)

        fire(0, es0_v, ed0_v, rows0_v, sem0)
        fire(1, es1_v, ed1_v, rows1_v, sem1)

        def process(j, es_v, ed_v, rows_v, sem):
            pltpu.make_async_copy(
                es_hbm.at[idxs2_v.at[j]], es_v, sem).wait()
            pltpu.make_async_copy(
                ed_hbm.at[idxd2_v.at[j]], ed_v, sem).wait()
            pltpu.make_async_copy(
                h2_hbm.at[idxs2_v.at[j]], rows_v, sem).wait()

            def vec(t, c2):
                sl = pl.ds(t * 16, 16)
                z = es_v[sl] + ed_v[sl]
                e = jnp.where(z > 0, z, 0.2 * z)
                ex = jnp.exp(e - shift)
                coef_v[sl] = ex * ew2_v[j, sl]
                ex1_v[sl] = ex
                return c2

            lax.fori_loop(0, CH // 16, vec, 0)

            def group(t, c2):
                gw = coef_v[pl.ds(t * 16, 16)]
                for l in range(16):
                    wl = _bcast(gw, l)
                    i = t * 16 + l
                    for jj in range(4):
                        rows_v[i, pl.ds(16 * jj, 16)] = (
                            rows_v[i, pl.ds(16 * jj, 16)] * wl)
                return c2

            lax.fori_loop(0, CH // 16, group, 0)
            pltpu.sync_copy(rows_v, agg_sh.at[idxd2_v.at[j]], add=True)
            pltpu.sync_copy(ex1_v, den_sh.at[idxd2_v.at[j]], add=True)
            nj = j + 2

            @pl.when(nj < NCHT)
            def _():
                fire(nj, es_v, ed_v, rows_v, sem)

        def body(m, carry):
            process(2 * m, es0_v, ed0_v, rows0_v, sem0)
            process(2 * m + 1, es1_v, ed1_v, rows1_v, sem1)
            return carry

        lax.fori_loop(0, NCHT // 2, body, 0)

    @pl.when(c == 0)
    def _():
        run(esk_hbm, edk_hbm, h2k_hbm, 0, 1)

    @pl.when(c == 1)
    def _():
        run(esv_hbm, edv_hbm, h2v_hbm, 2, 3)

    plsc.subcore_barrier()
    pltpu.sync_copy(agg_sh.at[pl.ds(s * RPT, RPT), :],
                    outa_hbm.at[c, pl.ds(s * RPT, RPT), :])
    pltpu.sync_copy(den_sh.at[pl.ds(s * RPT, RPT)],
                    outd_hbm.at[pl.ds(c * NP + s * RPT, RPT)])


def _pre_body(q_ref, k_ref, v_ref, dg0_ref, dg1_ref,
              wq_ref, wk1_ref, wv1_ref, wk2_ref, wv2_ref,
              alk_ref, ark_ref, alv_ref, arv_ref,
              hga_ref, hgb_ref, h2k_ref, h2v_ref,
              esk_ref, edk_ref, esv_ref, edv_ref, mx_ref):
    i = pl.program_id(0)
    deg = dg0_ref[0] + dg1_ref[0]
    no = lax.rsqrt(jnp.clip(deg[:, 0:1], 1.0, None))
    hq = jnp.dot(q_ref[...], wq_ref[...], preferred_element_type=_f32)
    hk1 = jnp.dot(k_ref[...], wk1_ref[...], preferred_element_type=_f32)
    hv1 = jnp.dot(k_ref[...], wv1_ref[...], preferred_element_type=_f32)
    hga_ref[...] = jnp.concatenate([hq, hk1[:, 0:32]], axis=1) * no
    hgb_ref[...] = jnp.concatenate([hk1[:, 32:64], hv1], axis=1) * no
    h2k = jnp.dot(v_ref[...], wk2_ref[...], preferred_element_type=_f32)
    h2v = jnp.dot(v_ref[...], wv2_ref[...], preferred_element_type=_f32)
    h2k_ref[...] = h2k
    h2v_ref[...] = h2v
    esk = jnp.dot(h2k, alk_ref[...], preferred_element_type=_f32)
    edk = jnp.dot(h2k, ark_ref[...], preferred_element_type=_f32)
    esv = jnp.dot(h2v, alv_ref[...], preferred_element_type=_f32)
    edv = jnp.dot(h2v, arv_ref[...], preferred_element_type=_f32)
    esk_ref[...] = esk
    edk_ref[...] = edk
    esv_ref[...] = esv
    edv_ref[...] = edv
    cur = jnp.stack([
        jnp.full((16,), jnp.max(esk), _f32),
        jnp.full((16,), jnp.max(edk), _f32),
        jnp.full((16,), jnp.max(esv), _f32),
        jnp.full((16,), jnp.max(edv), _f32),
        jnp.full((16,), jnp.max(esk), _f32),
        jnp.full((16,), jnp.max(edk), _f32),
        jnp.full((16,), jnp.max(esv), _f32),
        jnp.full((16,), jnp.max(edv), _f32),
    ])

    @pl.when(i == 0)
    def _():
        mx_ref[...] = cur

    @pl.when(i > 0)
    def _():
        mx_ref[...] = jnp.maximum(mx_ref[...], cur)


def _post_body(a0_ref, a1_ref, ak2_ref, av2_ref, dk_ref, dv_ref,
               dg0_ref, dg1_ref, bq_ref, bk1_ref, bv1_ref, bk2_ref, bv2_ref,
               wh_ref, bh_ref, y_ref):
    alo = a0_ref[0]
    ahi = a1_ref[0]
    deg = dg0_ref[0] + dg1_ref[0]
    ni = lax.rsqrt(jnp.clip(deg[:, 1:2], 1.0, None))
    q = alo[:, 0:64] * ni + bq_ref[...]
    k1 = (jnp.concatenate([alo[:, 64:96], ahi[:, 0:32]], axis=1) * ni
          + bk1_ref[...])
    v1 = ahi[:, 32:96] * ni + bv1_ref[...]
    denk = dk_ref[0]
    denv = dv_ref[0]
    k2 = ak2_ref[0] / jnp.where(denk > 0, denk, 1.0) + bk2_ref[...]
    v2 = av2_ref[0] / jnp.where(denv > 0, denv, 1.0) + bv2_ref[...]
    kv1 = jnp.sum(q * k1, axis=1, keepdims=True)
    kv2 = jnp.sum(q * k2, axis=1, keepdims=True)
    mx = jnp.maximum(kv1, kv2)
    e1 = jnp.exp(kv1 - mx)
    e2 = jnp.exp(kv2 - mx)
    ssum = e1 + e2
    out = (e1 / ssum) * v1 + (e2 / ssum) * v2
    wh = wh_ref[...]
    whf = wh[0:64] + wh[64:128] + wh[128:192] + wh[192:256]
    y_ref[...] = jnp.dot(out, whf, preferred_element_type=_f32) + bh_ref[...]


def _sc_mesh():
    return plsc.VectorSubcoreMesh(core_axis_name="c", subcore_axis_name="s",
                                  num_cores=NC, num_subcores=NS)


def kernel(Q, K, V, sg_edge_index, edfg_edge_index, sgFeat, edfgFeat,
           Wq, bq, Wk1, bk1, Wv1, bv1,
           Wk2, alk2, ark2, bk2, Wv2, alv2, arv2, bv2,
           Wh, bh):
    npad = EP - EE
    pad0 = jnp.zeros((npad,), jnp.int32)
    padn = jnp.full((npad,), NN, jnp.int32)
    padw = jnp.zeros((npad,), _f32)
    src_s = jnp.concatenate([sg_edge_index[0], pad0]).reshape(-1, CH)
    srcdeg_s = jnp.concatenate([sg_edge_index[0], padn]).reshape(-1, CH)
    dst_s = jnp.concatenate([sg_edge_index[1], padn]).reshape(-1, CH)
    sgw = jnp.concatenate([sgFeat, padw]).reshape(-1, CH)
    src_e = jnp.concatenate([edfg_edge_index[0], pad0]).reshape(-1, CH)
    dst_e = jnp.concatenate([edfg_edge_index[1], padn]).reshape(-1, CH)
    edw = jnp.concatenate([edfgFeat, padw]).reshape(-1, CH)

    w10 = jnp.tile(jnp.array([[1.0] + [0.0] * 15], _f32), (CH, 1))
    w01 = jnp.tile(jnp.array([[0.0, 1.0] + [0.0] * 14], _f32), (CH, 1))
    zeros16 = jnp.zeros((NP, 16), _f32)
    zeros96 = jnp.zeros((NP, 96), _f32)
    zeros64 = jnp.zeros((NP, 64), _f32)
    zeros1 = jnp.zeros((NP,), _f32)

    deg_call = pl.kernel(
        _deg_body,
        out_type=jax.ShapeDtypeStruct((NC, NP, 16), _f32),
        mesh=_sc_mesh(),
        compiler_params=pltpu.CompilerParams(use_tc_tiling_on_sc=False),
        scratch_types=[
            pltpu.VMEM((NCHD, CH), jnp.int32),
            pltpu.VMEM((NCHD, CH), jnp.int32),
            pltpu.VMEM((CH, 16), _f32),
            pltpu.VMEM((CH, 16), _f32),
            pltpu.VMEM_SHARED((NP, 16), _f32),
        ],
    )
    degp = deg_call(srcdeg_s, dst_s, w10, w01, zeros16)

    BN = 1000
    grid = (NN // BN,)

    pre_call = pl.pallas_call(
        _pre_body,
        grid=grid,
        in_specs=[
            pl.BlockSpec((BN, 256), lambda i: (i, 0)),
            pl.BlockSpec((BN, 256), lambda i: (i, 0)),
            pl.BlockSpec((BN, 256), lambda i: (i, 0)),
            pl.BlockSpec((1, BN, 16), lambda i: (0, i, 0)),
            pl.BlockSpec((1, BN, 16), lambda i: (1, i, 0)),
            pl.BlockSpec((256, 64), lambda i: (0, 0)),
            pl.BlockSpec((256, 64), lambda i: (0, 0)),
            pl.BlockSpec((256, 64), lambda i: (0, 0)),
            pl.BlockSpec((256, 64), lambda i: (0, 0)),
            pl.BlockSpec((256, 64), lambda i: (0, 0)),
            pl.BlockSpec((64, 1), lambda i: (0, 0)),
            pl.BlockSpec((64, 1), lambda i: (0, 0)),
            pl.BlockSpec((64, 1), lambda i: (0, 0)),
            pl.BlockSpec((64, 1), lambda i: (0, 0)),
        ],
        out_specs=[
            pl.BlockSpec((BN, 96), lambda i: (i, 0)),
            pl.BlockSpec((BN, 96), lambda i: (i, 0)),
            pl.BlockSpec((BN, 64), lambda i: (i, 0)),
            pl.BlockSpec((BN, 64), lambda i: (i, 0)),
            pl.BlockSpec((BN, 1), lambda i: (i, 0)),
            pl.BlockSpec((BN, 1), lambda i: (i, 0)),
            pl.BlockSpec((BN, 1), lambda i: (i, 0)),
            pl.BlockSpec((BN, 1), lambda i: (i, 0)),
            pl.BlockSpec((8, 16), lambda i: (0, 0)),
        ],
        out_shape=[
            jax.ShapeDtypeStruct((NN, 96), _f32),
            jax.ShapeDtypeStruct((NN, 96), _f32),
            jax.ShapeDtypeStruct((NN, 64), _f32),
            jax.ShapeDtypeStruct((NN, 64), _f32),
            jax.ShapeDtypeStruct((NN, 1), _f32),
            jax.ShapeDtypeStruct((NN, 1), _f32),
            jax.ShapeDtypeStruct((NN, 1), _f32),
            jax.ShapeDtypeStruct((NN, 1), _f32),
            jax.ShapeDtypeStruct((8, 16), _f32),
        ],
    )
    hga, hgb, h2k, h2v, esk, edk, esv, edv, mx = pre_call(
        Q, K, V, degp, degp, Wq, Wk1, Wv1, Wk2, Wv2,
        alk2.reshape(64, 1), ark2.reshape(64, 1),
        alv2.reshape(64, 1), arv2.reshape(64, 1))

    gcn_call = pl.kernel(
        _gcn_body,
        out_type=jax.ShapeDtypeStruct((NC, NP, 96), _f32),
        mesh=_sc_mesh(),
        compiler_params=pltpu.CompilerParams(use_tc_tiling_on_sc=False),
        scratch_types=[
            pltpu.VMEM((NCHT, CH), jnp.int32),
            pltpu.VMEM((NCHT, CH), jnp.int32),
            pltpu.VMEM((NCHT, CH), _f32),
            pltpu.VMEM((CH, 96), _f32),
            pltpu.VMEM((CH, 96), _f32),
            pltpu.SemaphoreType.DMA,
            pltpu.SemaphoreType.DMA,
            pltpu.VMEM_SHARED((NP, 96), _f32),
        ],
    )
    aggp = gcn_call(src_s, dst_s, sgw, hga, hgb, zeros96)

    gat_call = pl.kernel(
        _gat_body,
        out_type=(
            jax.ShapeDtypeStruct((NC, NP, 64), _f32),
            jax.ShapeDtypeStruct((NC * NP,), _f32),
        ),
        mesh=_sc_mesh(),
        compiler_params=pltpu.CompilerParams(use_tc_tiling_on_sc=False),
        scratch_types=[
            pltpu.VMEM((NCHT, CH), jnp.int32),
            pltpu.VMEM((NCHT, CH), jnp.int32),
            pltpu.VMEM((NCHT, CH), _f32),
            pltpu.VMEM((CH,), _f32),
            pltpu.VMEM((CH,), _f32),
            pltpu.VMEM((CH,), _f32),
            pltpu.VMEM((CH,), _f32),
            pltpu.VMEM((CH, 64), _f32),
            pltpu.VMEM((CH, 64), _f32),
            pltpu.VMEM((CH,), _f32),
            pltpu.VMEM((CH,), _f32),
            pltpu.VMEM((8, 16), _f32),
            pltpu.SemaphoreType.DMA,
            pltpu.SemaphoreType.DMA,
            pltpu.VMEM_SHARED((NP, 64), _f32),
            pltpu.VMEM_SHARED((NP,), _f32),
        ],
    )
    padt = jnp.zeros((NP - NN,), _f32)
    edk_p = jnp.concatenate([edk.reshape(NN), padt])
    edv_p = jnp.concatenate([edv.reshape(NN), padt])
    agg2, den_flat = gat_call(src_e, dst_e, edw, h2k, h2v,
                              esk.reshape(NN), edk_p,
                              esv.reshape(NN), edv_p, mx,
                              zeros64, zeros1)
    den = den_flat.reshape(NC, NP, 1)

    post_call = pl.pallas_call(
        _post_body,
        grid=grid,
        in_specs=[
            pl.BlockSpec((1, BN, 96), lambda i: (0, i, 0)),
            pl.BlockSpec((1, BN, 96), lambda i: (1, i, 0)),
            pl.BlockSpec((1, BN, 64), lambda i: (0, i, 0)),
            pl.BlockSpec((1, BN, 64), lambda i: (1, i, 0)),
            pl.BlockSpec((1, BN, 1), lambda i: (0, i, 0)),
            pl.BlockSpec((1, BN, 1), lambda i: (1, i, 0)),
            pl.BlockSpec((1, BN, 16), lambda i: (0, i, 0)),
            pl.BlockSpec((1, BN, 16), lambda i: (1, i, 0)),
            pl.BlockSpec((1, 64), lambda i: (0, 0)),
            pl.BlockSpec((1, 64), lambda i: (0, 0)),
            pl.BlockSpec((1, 64), lambda i: (0, 0)),
            pl.BlockSpec((1, 64), lambda i: (0, 0)),
            pl.BlockSpec((1, 64), lambda i: (0, 0)),
            pl.BlockSpec((256, 256), lambda i: (0, 0)),
            pl.BlockSpec((1, 256), lambda i: (0, 0)),
        ],
        out_specs=pl.BlockSpec((BN, 256), lambda i: (i, 0)),
        out_shape=jax.ShapeDtypeStruct((NN, 256), _f32),
    )
    y = post_call(aggp, aggp, agg2, agg2, den, den, degp, degp,
                  bq.reshape(1, 64), bk1.reshape(1, 64), bv1.reshape(1, 64),
                  bk2.reshape(1, 64), bv2.reshape(1, 64),
                  Wh, bh.reshape(1, 256))
    return y
